# baseline (speedup 1.0000x reference)
import jax
import jax.numpy as jnp
from jax import lax
from jax.experimental import pallas as pl
from jax.experimental.pallas import tpu as pltpu
from jax.experimental.pallas import tpu_sc as plsc

BATCH = 16384
FACTORS = 64
NC = 2
NS = 16
NW = NC * NS
B_PER_W = BATCH // NW
CHUNK = 128
NCHUNK = B_PER_W // CHUNK
LANES = 16
NGROUP = B_PER_W // LANES


def _body(user_hbm, item_hbm, uw_hbm, iw_hbm, out_hbm,
          idx_u, idx_i, rows_u, rows_i, out_v, sem):
    wid = lax.axis_index("s") * NC + lax.axis_index("c")

    pltpu.sync_copy(user_hbm.at[pl.ds(wid * NCHUNK, NCHUNK)], idx_u)
    pltpu.sync_copy(item_hbm.at[pl.ds(wid * NCHUNK, NCHUNK)], idx_i)

    copies = []
    for j in range(NCHUNK):
        copies.append(pltpu.async_copy(
            uw_hbm.at[idx_u.at[j]], rows_u.at[pl.ds(j * CHUNK, CHUNK)], sem))
        copies.append(pltpu.async_copy(
            iw_hbm.at[idx_i.at[j]], rows_i.at[pl.ds(j * CHUNK, CHUNK)], sem))
    for c in copies:
        c.wait()

    def group(g, carry):
        row_ids = g * LANES + lax.iota(jnp.int32, LANES)
        acc = None
        for f in range(FACTORS):
            col = jnp.full((LANES,), f, dtype=jnp.int32)
            gu = plsc.load_gather(rows_u, [row_ids, col])
            gi = plsc.load_gather(rows_i, [row_ids, col])
            acc = gu * gi if acc is None else acc + gu * gi
        out_v[g, :] = acc
        return carry

    lax.fori_loop(0, NGROUP, group, 0)

    pltpu.sync_copy(out_v, out_hbm.at[pl.ds(wid * NGROUP, NGROUP)])


@jax.jit
def _pointmf_sc(user2d, item2d, uw, iw):
    mesh = plsc.VectorSubcoreMesh(core_axis_name="c", subcore_axis_name="s")
    return pl.kernel(
        _body,
        mesh=mesh,
        compiler_params=pltpu.CompilerParams(
            needs_layout_passes=False, use_tc_tiling_on_sc=False),
        out_type=jax.ShapeDtypeStruct((BATCH // LANES, LANES), jnp.float32),
        scratch_types=[
            pltpu.VMEM((NCHUNK, CHUNK), jnp.int32),
            pltpu.VMEM((NCHUNK, CHUNK), jnp.int32),
            pltpu.VMEM((B_PER_W, FACTORS), jnp.float32),
            pltpu.VMEM((B_PER_W, FACTORS), jnp.float32),
            pltpu.VMEM((NGROUP, LANES), jnp.float32),
            pltpu.SemaphoreType.DMA,
        ],
    )(user2d, item2d, uw, iw)


def kernel(user, item, context, embed_user_w, embed_item_w):
    del context
    user2d = user.astype(jnp.int32).reshape(BATCH // CHUNK, CHUNK)
    item2d = item.astype(jnp.int32).reshape(BATCH // CHUNK, CHUNK)
    out = _pointmf_sc(user2d, item2d, embed_user_w, embed_item_w)
    return out.reshape(BATCH)
